# baseline (device time: 62414 ns/iter reference)
import jax
import jax.numpy as jnp
from jax import lax
from jax.experimental import pallas as pl
from jax.experimental.pallas import tpu as pltpu

N_DEV = 4
SEQ = 2048
HALO = 128
EXT = SEQ + 2 * HALO
HQ, DH = 8, 128
DM = HQ * DH
BQ = 128
BK = BQ + 2 * HALO
SCALE = 0.08838834764831843
GSEQ = N_DEV * SEQ


def _body(x_ref, wq_ref, k_ref, v_ref, wo_ref, out_ref,
          kbuf, vbuf, qbuf, cbuf, wobuf, kst, vst,
          copy_sems, send_sems, recv_sems):
    my = lax.axis_index("i")
    has_left = my > 0
    has_right = my < N_DEV - 1
    left = jnp.maximum(my - 1, 0)
    right = jnp.minimum(my + 1, N_DEV - 1)

    kv_copies = []
    for h in range(HQ):
        ck = pltpu.make_async_copy(
            k_ref.at[0, :, h, :], kst.at[h], copy_sems.at[h])
        cv = pltpu.make_async_copy(
            v_ref.at[0, :, h, :], vst.at[h], copy_sems.at[HQ + h])
        ck.start()
        cv.start()
        kv_copies.append((ck, cv))

    qbuf[:, :] = (jnp.dot(
        x_ref[0, :, :].astype(jnp.bfloat16), wq_ref[:, :].astype(jnp.bfloat16),
        preferred_element_type=jnp.float32) * SCALE).astype(jnp.bfloat16)
    wobuf[:, :] = wo_ref[:, :].astype(jnp.bfloat16)

    for h in range(HQ):
        ck, cv = kv_copies[h]
        c0 = h * DH
        ck.wait()
        kbuf[HALO:HALO + SEQ, c0:c0 + DH] = kst[h].astype(jnp.bfloat16)
        cv.wait()
        vbuf[HALO:HALO + SEQ, c0:c0 + DH] = vst[h].astype(jnp.bfloat16)
    zeros = jnp.zeros((HALO, DM), jnp.bfloat16)
    kbuf[0:HALO, :] = zeros
    vbuf[0:HALO, :] = zeros
    kbuf[HALO + SEQ:EXT, :] = zeros
    vbuf[HALO + SEQ:EXT, :] = zeros

    barrier_sem = pltpu.get_barrier_semaphore()

    @pl.when(has_left)
    def _():
        pl.semaphore_signal(barrier_sem, inc=1, device_id=(left,),
                            device_id_type=pl.DeviceIdType.MESH)

    @pl.when(has_right)
    def _():
        pl.semaphore_signal(barrier_sem, inc=1, device_id=(right,),
                            device_id_type=pl.DeviceIdType.MESH)

    @pl.when(has_left)
    def _():
        pl.semaphore_wait(barrier_sem, 1)

    @pl.when(has_right)
    def _():
        pl.semaphore_wait(barrier_sem, 1)

    send_right_k = pltpu.make_async_remote_copy(
        src_ref=kbuf.at[pl.ds(SEQ, HALO)], dst_ref=kbuf.at[pl.ds(0, HALO)],
        send_sem=send_sems.at[0], recv_sem=recv_sems.at[0],
        device_id=(right,), device_id_type=pl.DeviceIdType.MESH)
    send_right_v = pltpu.make_async_remote_copy(
        src_ref=vbuf.at[pl.ds(SEQ, HALO)], dst_ref=vbuf.at[pl.ds(0, HALO)],
        send_sem=send_sems.at[1], recv_sem=recv_sems.at[1],
        device_id=(right,), device_id_type=pl.DeviceIdType.MESH)
    send_left_k = pltpu.make_async_remote_copy(
        src_ref=kbuf.at[pl.ds(HALO, HALO)],
        dst_ref=kbuf.at[pl.ds(HALO + SEQ, HALO)],
        send_sem=send_sems.at[2], recv_sem=recv_sems.at[2],
        device_id=(left,), device_id_type=pl.DeviceIdType.MESH)
    send_left_v = pltpu.make_async_remote_copy(
        src_ref=vbuf.at[pl.ds(HALO, HALO)],
        dst_ref=vbuf.at[pl.ds(HALO + SEQ, HALO)],
        send_sem=send_sems.at[3], recv_sem=recv_sems.at[3],
        device_id=(left,), device_id_type=pl.DeviceIdType.MESH)

    @pl.when(has_right)
    def _():
        send_right_k.start()
        send_right_v.start()

    @pl.when(has_left)
    def _():
        send_left_k.start()
        send_left_v.start()

    ii = lax.broadcasted_iota(jnp.int32, (BQ, BK), 0)
    jj = lax.broadcasted_iota(jnp.int32, (BQ, BK), 1)
    bandf = ((jj >= ii) & (jj <= ii + 2 * HALO)).astype(jnp.float32)
    jrow = lax.broadcasted_iota(jnp.int32, (1, BK), 1)

    def qblock(qb, _):
        q0 = qb * BQ
        kpos = my * SEQ - HALO + q0 + jrow
        maskf = bandf * ((kpos >= 0) & (kpos < GSEQ)).astype(jnp.float32)
        for h in range(HQ):
            c0 = h * DH
            qblk = qbuf[pl.ds(q0, BQ), c0:c0 + DH]
            kblk = kbuf[pl.ds(q0, BK), c0:c0 + DH]
            s = lax.dot_general(
                qblk, kblk, (((1,), (1,)), ((), ())),
                preferred_element_type=jnp.float32)
            w = jnp.exp(s) * maskf
            rec = 1.0 / jnp.sum(w, axis=1, keepdims=True)
            vblk = vbuf[pl.ds(q0, BK), c0:c0 + DH]
            ctx = lax.dot_general(
                w.astype(jnp.bfloat16), vblk, (((1,), (0,)), ((), ())),
                preferred_element_type=jnp.float32)
            cbuf[:, c0:c0 + DH] = (ctx * rec).astype(jnp.bfloat16)
        out_ref[0, pl.ds(q0, BQ), :] = jnp.dot(
            cbuf[:, :], wobuf[:, :], preferred_element_type=jnp.float32)
        return 0

    lax.fori_loop(1, SEQ // BQ - 1, qblock, 0)

    @pl.when(has_left)
    def _():
        send_right_k.wait_recv()
        send_right_v.wait_recv()
        send_left_k.wait_send()
        send_left_v.wait_send()

    @pl.when(has_right)
    def _():
        send_left_k.wait_recv()
        send_left_v.wait_recv()
        send_right_k.wait_send()
        send_right_v.wait_send()

    qblock(0, 0)
    qblock(SEQ // BQ - 1, 0)


def kernel(x, Wq, K_ext, V_ext, Wo):
    return pl.pallas_call(
        _body,
        out_shape=jax.ShapeDtypeStruct((1, SEQ, DM), jnp.float32),
        in_specs=[
            pl.BlockSpec(memory_space=pltpu.VMEM),
            pl.BlockSpec(memory_space=pltpu.VMEM),
            pl.BlockSpec(memory_space=pltpu.MemorySpace.HBM),
            pl.BlockSpec(memory_space=pltpu.MemorySpace.HBM),
            pl.BlockSpec(memory_space=pltpu.VMEM),
        ],
        out_specs=pl.BlockSpec(memory_space=pltpu.VMEM),
        scratch_shapes=[
            pltpu.VMEM((EXT, DM), jnp.bfloat16),
            pltpu.VMEM((EXT, DM), jnp.bfloat16),
            pltpu.VMEM((SEQ, DM), jnp.bfloat16),
            pltpu.VMEM((BQ, DM), jnp.bfloat16),
            pltpu.VMEM((DM, DM), jnp.bfloat16),
            pltpu.VMEM((HQ, SEQ, DH), jnp.float32),
            pltpu.VMEM((HQ, SEQ, DH), jnp.float32),
            pltpu.SemaphoreType.DMA((2 * HQ,)),
            pltpu.SemaphoreType.DMA((4,)),
            pltpu.SemaphoreType.DMA((4,)),
        ],
        compiler_params=pltpu.CompilerParams(
            collective_id=0, vmem_limit_bytes=100 * 1024 * 1024),
    )(x, Wq, K_ext, V_ext, Wo)


# device time: 51576 ns/iter; 1.2101x vs baseline; 1.2101x over previous
import jax
import jax.numpy as jnp
from jax import lax
from jax.experimental import pallas as pl
from jax.experimental.pallas import tpu as pltpu

N_DEV = 4
SEQ = 2048
HALO = 128
EXT = SEQ + 2 * HALO
HQ, DH = 8, 128
DM = HQ * DH
BQ = 256
BK = BQ + 2 * HALO
SCALE = 0.08838834764831843
GSEQ = N_DEV * SEQ


def _body(x_ref, wq_ref, k_ref, v_ref, wo_ref, out_ref,
          kbuf, vbuf, qbuf, cbuf, wobuf, kst, vst,
          copy_sems, send_sems, recv_sems):
    my = lax.axis_index("i")
    has_left = my > 0
    has_right = my < N_DEV - 1
    left = jnp.maximum(my - 1, 0)
    right = jnp.minimum(my + 1, N_DEV - 1)

    kv_copies = []
    for h in range(HQ):
        ck = pltpu.make_async_copy(
            k_ref.at[0, :, h, :], kst.at[h], copy_sems.at[h])
        cv = pltpu.make_async_copy(
            v_ref.at[0, :, h, :], vst.at[h], copy_sems.at[HQ + h])
        ck.start()
        cv.start()
        kv_copies.append((ck, cv))

    qbuf[:, :] = (jnp.dot(
        x_ref[0, :, :].astype(jnp.bfloat16), wq_ref[:, :].astype(jnp.bfloat16),
        preferred_element_type=jnp.float32) * SCALE).astype(jnp.bfloat16)
    wobuf[:, :] = wo_ref[:, :].astype(jnp.bfloat16)

    for h in range(HQ):
        ck, cv = kv_copies[h]
        c0 = h * DH
        ck.wait()
        kbuf[HALO:HALO + SEQ, c0:c0 + DH] = kst[h].astype(jnp.bfloat16)
        cv.wait()
        vbuf[HALO:HALO + SEQ, c0:c0 + DH] = vst[h].astype(jnp.bfloat16)
    zeros = jnp.zeros((HALO, DM), jnp.bfloat16)
    kbuf[0:HALO, :] = zeros
    vbuf[0:HALO, :] = zeros
    kbuf[HALO + SEQ:EXT, :] = zeros
    vbuf[HALO + SEQ:EXT, :] = zeros

    barrier_sem = pltpu.get_barrier_semaphore()

    @pl.when(has_left)
    def _():
        pl.semaphore_signal(barrier_sem, inc=1, device_id=(left,),
                            device_id_type=pl.DeviceIdType.MESH)

    @pl.when(has_right)
    def _():
        pl.semaphore_signal(barrier_sem, inc=1, device_id=(right,),
                            device_id_type=pl.DeviceIdType.MESH)

    @pl.when(has_left)
    def _():
        pl.semaphore_wait(barrier_sem, 1)

    @pl.when(has_right)
    def _():
        pl.semaphore_wait(barrier_sem, 1)

    send_right_k = pltpu.make_async_remote_copy(
        src_ref=kbuf.at[pl.ds(SEQ, HALO)], dst_ref=kbuf.at[pl.ds(0, HALO)],
        send_sem=send_sems.at[0], recv_sem=recv_sems.at[0],
        device_id=(right,), device_id_type=pl.DeviceIdType.MESH)
    send_right_v = pltpu.make_async_remote_copy(
        src_ref=vbuf.at[pl.ds(SEQ, HALO)], dst_ref=vbuf.at[pl.ds(0, HALO)],
        send_sem=send_sems.at[1], recv_sem=recv_sems.at[1],
        device_id=(right,), device_id_type=pl.DeviceIdType.MESH)
    send_left_k = pltpu.make_async_remote_copy(
        src_ref=kbuf.at[pl.ds(HALO, HALO)],
        dst_ref=kbuf.at[pl.ds(HALO + SEQ, HALO)],
        send_sem=send_sems.at[2], recv_sem=recv_sems.at[2],
        device_id=(left,), device_id_type=pl.DeviceIdType.MESH)
    send_left_v = pltpu.make_async_remote_copy(
        src_ref=vbuf.at[pl.ds(HALO, HALO)],
        dst_ref=vbuf.at[pl.ds(HALO + SEQ, HALO)],
        send_sem=send_sems.at[3], recv_sem=recv_sems.at[3],
        device_id=(left,), device_id_type=pl.DeviceIdType.MESH)

    @pl.when(has_right)
    def _():
        send_right_k.start()
        send_right_v.start()

    @pl.when(has_left)
    def _():
        send_left_k.start()
        send_left_v.start()

    ii = lax.broadcasted_iota(jnp.int32, (BQ, BK), 0)
    jj = lax.broadcasted_iota(jnp.int32, (BQ, BK), 1)
    bandf = ((jj >= ii) & (jj <= ii + 2 * HALO)).astype(jnp.float32)
    jrow = lax.broadcasted_iota(jnp.int32, (1, BK), 1)

    def qblock(qb, _):
        q0 = qb * BQ
        kpos = my * SEQ - HALO + q0 + jrow
        maskf = bandf * ((kpos >= 0) & (kpos < GSEQ)).astype(jnp.float32)
        for h in range(HQ):
            c0 = h * DH
            qblk = qbuf[pl.ds(q0, BQ), c0:c0 + DH]
            kblk = kbuf[pl.ds(q0, BK), c0:c0 + DH]
            s = lax.dot_general(
                qblk, kblk, (((1,), (1,)), ((), ())),
                preferred_element_type=jnp.float32)
            w = jnp.exp(s) * maskf
            rec = 1.0 / jnp.sum(w, axis=1, keepdims=True)
            vblk = vbuf[pl.ds(q0, BK), c0:c0 + DH]
            ctx = lax.dot_general(
                w.astype(jnp.bfloat16), vblk, (((1,), (0,)), ((), ())),
                preferred_element_type=jnp.float32)
            cbuf[:, c0:c0 + DH] = (ctx * rec).astype(jnp.bfloat16)
        out_ref[0, pl.ds(q0, BQ), :] = jnp.dot(
            cbuf[:, :], wobuf[:, :], preferred_element_type=jnp.float32)
        return 0

    lax.fori_loop(1, SEQ // BQ - 1, qblock, 0)

    @pl.when(has_left)
    def _():
        send_right_k.wait_recv()
        send_right_v.wait_recv()
        send_left_k.wait_send()
        send_left_v.wait_send()

    @pl.when(has_right)
    def _():
        send_left_k.wait_recv()
        send_left_v.wait_recv()
        send_right_k.wait_send()
        send_right_v.wait_send()

    qblock(0, 0)
    qblock(SEQ // BQ - 1, 0)


def kernel(x, Wq, K_ext, V_ext, Wo):
    return pl.pallas_call(
        _body,
        out_shape=jax.ShapeDtypeStruct((1, SEQ, DM), jnp.float32),
        in_specs=[
            pl.BlockSpec(memory_space=pltpu.VMEM),
            pl.BlockSpec(memory_space=pltpu.VMEM),
            pl.BlockSpec(memory_space=pltpu.MemorySpace.HBM),
            pl.BlockSpec(memory_space=pltpu.MemorySpace.HBM),
            pl.BlockSpec(memory_space=pltpu.VMEM),
        ],
        out_specs=pl.BlockSpec(memory_space=pltpu.VMEM),
        scratch_shapes=[
            pltpu.VMEM((EXT, DM), jnp.bfloat16),
            pltpu.VMEM((EXT, DM), jnp.bfloat16),
            pltpu.VMEM((SEQ, DM), jnp.bfloat16),
            pltpu.VMEM((BQ, DM), jnp.bfloat16),
            pltpu.VMEM((DM, DM), jnp.bfloat16),
            pltpu.VMEM((HQ, SEQ, DH), jnp.float32),
            pltpu.VMEM((HQ, SEQ, DH), jnp.float32),
            pltpu.SemaphoreType.DMA((2 * HQ,)),
            pltpu.SemaphoreType.DMA((4,)),
            pltpu.SemaphoreType.DMA((4,)),
        ],
        compiler_params=pltpu.CompilerParams(
            collective_id=0, vmem_limit_bytes=100 * 1024 * 1024),
    )(x, Wq, K_ext, V_ext, Wo)
